# baseline (speedup 1.0000x reference)
import jax
import jax.numpy as jnp
from jax import lax
from jax.experimental import pallas as pl
from jax.experimental.pallas import tpu as pltpu
from jax.experimental.pallas import tpu_sc as plsc

B = 128
N = 8192
K = 5
L = 16
NC = 2
NS = 16
NW = NC * NS
ROWS_PER_W = B // NW
NV = N // L
NEG_INF = float("-inf")


def _row_correct(pred_v, targ_v, acc):
    @plsc.parallel_loop(
        0,
        NV,
        unroll=16,
        carry=tuple(jnp.full((L,), NEG_INF, jnp.float32) for _ in range(K)),
    )
    def ms(i, ms):
        v = pred_v[pl.ds(i * L, L)]
        out = []
        for m in ms:
            hi = jnp.maximum(m, v)
            v = jnp.minimum(m, v)
            out.append(hi)
        return tuple(out)

    def body_t(_, carry):
        c1, c2, c3, c4, c5, accn, cgt, t = carry
        cmax = jnp.maximum(
            jnp.maximum(jnp.maximum(c1, c2), jnp.maximum(c3, c4)), c5
        )
        g = jnp.max(cmax)
        eqs = (
            (c1 == g).astype(jnp.int32)
            + (c2 == g).astype(jnp.int32)
            + (c3 == g).astype(jnp.int32)
            + (c4 == g).astype(jnp.int32)
            + (c5 == g).astype(jnp.int32)
        )
        cnt = jnp.sum(eqs)
        nd = accn < K
        t = jnp.where(nd, g, t)
        cgt = jnp.where(nd, accn, cgt)
        accn = jnp.where(nd, accn + cnt, accn)
        c1 = jnp.where(c1 == g, NEG_INF, c1)
        c2 = jnp.where(c2 == g, NEG_INF, c2)
        c3 = jnp.where(c3 == g, NEG_INF, c3)
        c4 = jnp.where(c4 == g, NEG_INF, c4)
        c5 = jnp.where(c5 == g, NEG_INF, c5)
        return c1, c2, c3, c4, c5, accn, cgt, t

    carry = (*ms, jnp.int32(0), jnp.int32(0), jnp.float32(0))
    carry = lax.fori_loop(0, K, body_t, carry)
    cgt, t = carry[6], carry[7]
    r = K - cgt

    zeros = jnp.zeros((L,), jnp.int32)

    @plsc.parallel_loop(0, NV, unroll=16, carry=(zeros, zeros))
    def accs(i, carry):
        acc_ge, cnt_eq = carry
        v = pred_v[pl.ds(i * L, L)]
        tv = targ_v[pl.ds(i * L, L)]
        acc_ge = acc_ge + jnp.where(v >= t, tv, 0)
        cnt_eq = cnt_eq + (v == t).astype(jnp.int32)
        return acc_ge, cnt_eq

    acc_ge, cnt_eq = accs
    tot_eq = jnp.sum(cnt_eq)

    def tie_scan(_):
        def body_c(i, carry):
            acc_gt, stie, taken = carry
            v = pred_v[pl.ds(i * L, L)]
            tv = targ_v[pl.ds(i * L, L)]
            acc_gt = acc_gt + jnp.where(v > t, tv, 0)
            eqi = (v == t).astype(jnp.int32)
            pref = jnp.cumsum(eqi)
            take = jnp.logical_and(v == t, (taken + pref) <= r)
            stie = stie + jnp.sum(jnp.where(take, tv, 0))
            taken = taken + jnp.sum(eqi)
            return acc_gt, stie, taken

        acc_gt, stie, _ = lax.fori_loop(
            0, NV, body_c, (zeros, jnp.int32(0), jnp.int32(0))
        )
        lane0 = lax.iota(jnp.int32, L) == 0
        return acc_gt + jnp.where(lane0, stie, 0)

    def ge_whole(_):
        return acc_ge

    return acc + lax.cond(tot_eq == r, ge_whole, tie_scan, 0)


def _sc_body(pred_hbm, targ_hbm, out_hbm, pred0, pred1, targ0, targ1, out_v,
             sp0, sp1, st0, st1):
    wid = lax.axis_index("s") * NC + lax.axis_index("c")
    row0 = wid * ROWS_PER_W
    preds = (pred0, pred1)
    targs = (targ0, targ1)
    sems_p = (sp0, sp1)
    sems_t = (st0, st1)
    cp = pltpu.async_copy(pred_hbm.at[row0], pred0, sp0)
    ct = pltpu.async_copy(targ_hbm.at[row0], targ0, st0)
    acc = jnp.zeros((L,), jnp.int32)
    for j in range(ROWS_PER_W):
        b = j % 2
        nb = (j + 1) % 2
        if j + 1 < ROWS_PER_W:
            ncp = pltpu.async_copy(
                pred_hbm.at[row0 + j + 1], preds[nb], sems_p[nb]
            )
            nct = pltpu.async_copy(
                targ_hbm.at[row0 + j + 1], targs[nb], sems_t[nb]
            )
        cp.wait()
        ct.wait()
        acc = _row_correct(preds[b], targs[b], acc)
        if j + 1 < ROWS_PER_W:
            cp, ct = ncp, nct
    out_v[...] = acc
    pltpu.sync_copy(out_v, out_hbm.at[wid])


@jax.jit
def kernel(pred, target):
    mesh = plsc.VectorSubcoreMesh(
        core_axis_name="c", subcore_axis_name="s", num_cores=NC, num_subcores=NS
    )
    partials = pl.kernel(
        _sc_body,
        out_type=jax.ShapeDtypeStruct((NW, L), jnp.int32),
        mesh=mesh,
        scratch_types=[
            pltpu.VMEM((N,), jnp.float32),
            pltpu.VMEM((N,), jnp.float32),
            pltpu.VMEM((N,), jnp.int32),
            pltpu.VMEM((N,), jnp.int32),
            pltpu.VMEM((L,), jnp.int32),
            pltpu.SemaphoreType.DMA,
            pltpu.SemaphoreType.DMA,
            pltpu.SemaphoreType.DMA,
            pltpu.SemaphoreType.DMA,
        ],
        compiler_params=pltpu.CompilerParams(needs_layout_passes=False),
    )(pred, target)
    correct = jnp.sum(partials).astype(jnp.float32)
    return correct * jnp.float32(100.0 / (B * K))


# speedup vs baseline: 1.2122x; 1.2122x over previous
import jax
import jax.numpy as jnp
from jax import lax
from jax.experimental import pallas as pl
from jax.experimental.pallas import tpu as pltpu
from jax.experimental.pallas import tpu_sc as plsc

B = 128
N = 8192
K = 5
L = 16
NC = 2
NS = 16
NW = NC * NS
ROWS_PER_W = B // NW
NV = N // L
NEG_INF = float("-inf")


def _row_correct(pred_v, targ_v, acc):
    @plsc.parallel_loop(
        0,
        NV,
        unroll=4,
        carry=tuple(jnp.full((L,), NEG_INF, jnp.float32) for _ in range(K)),
    )
    def ms(i, ms):
        v = pred_v[pl.ds(i * L, L)]
        out = []
        for m in ms:
            hi = jnp.maximum(m, v)
            v = jnp.minimum(m, v)
            out.append(hi)
        return tuple(out)

    def body_t(_, carry):
        c1, c2, c3, c4, c5, accn, cgt, t = carry
        cmax = jnp.maximum(
            jnp.maximum(jnp.maximum(c1, c2), jnp.maximum(c3, c4)), c5
        )
        g = jnp.max(cmax)
        eqs = (
            (c1 == g).astype(jnp.int32)
            + (c2 == g).astype(jnp.int32)
            + (c3 == g).astype(jnp.int32)
            + (c4 == g).astype(jnp.int32)
            + (c5 == g).astype(jnp.int32)
        )
        cnt = jnp.sum(eqs)
        nd = accn < K
        t = jnp.where(nd, g, t)
        cgt = jnp.where(nd, accn, cgt)
        accn = jnp.where(nd, accn + cnt, accn)
        c1 = jnp.where(c1 == g, NEG_INF, c1)
        c2 = jnp.where(c2 == g, NEG_INF, c2)
        c3 = jnp.where(c3 == g, NEG_INF, c3)
        c4 = jnp.where(c4 == g, NEG_INF, c4)
        c5 = jnp.where(c5 == g, NEG_INF, c5)
        return c1, c2, c3, c4, c5, accn, cgt, t

    carry = (*ms, jnp.int32(0), jnp.int32(0), jnp.float32(0))
    carry = lax.fori_loop(0, K, body_t, carry)
    cgt, t = carry[6], carry[7]
    r = K - cgt

    zeros = jnp.zeros((L,), jnp.int32)

    @plsc.parallel_loop(0, NV, unroll=4, carry=(zeros, zeros))
    def accs(i, carry):
        acc_ge, cnt_eq = carry
        v = pred_v[pl.ds(i * L, L)]
        tv = targ_v[pl.ds(i * L, L)]
        acc_ge = acc_ge + jnp.where(v >= t, tv, 0)
        cnt_eq = cnt_eq + (v == t).astype(jnp.int32)
        return acc_ge, cnt_eq

    acc_ge, cnt_eq = accs
    tot_eq = jnp.sum(cnt_eq)

    def tie_scan(_):
        def body_c(i, carry):
            acc_gt, stie, taken = carry
            v = pred_v[pl.ds(i * L, L)]
            tv = targ_v[pl.ds(i * L, L)]
            acc_gt = acc_gt + jnp.where(v > t, tv, 0)
            eqi = (v == t).astype(jnp.int32)
            pref = jnp.cumsum(eqi)
            take = jnp.logical_and(v == t, (taken + pref) <= r)
            stie = stie + jnp.sum(jnp.where(take, tv, 0))
            taken = taken + jnp.sum(eqi)
            return acc_gt, stie, taken

        acc_gt, stie, _ = lax.fori_loop(
            0, NV, body_c, (zeros, jnp.int32(0), jnp.int32(0))
        )
        lane0 = lax.iota(jnp.int32, L) == 0
        return acc_gt + jnp.where(lane0, stie, 0)

    def ge_whole(_):
        return acc_ge

    return acc + lax.cond(tot_eq == r, ge_whole, tie_scan, 0)


def _sc_body(pred_hbm, targ_hbm, out_hbm, pred0, pred1, targ0, targ1, out_v,
             sp0, sp1, st0, st1):
    wid = lax.axis_index("s") * NC + lax.axis_index("c")
    row0 = wid * ROWS_PER_W
    preds = (pred0, pred1)
    targs = (targ0, targ1)
    sems_p = (sp0, sp1)
    sems_t = (st0, st1)
    cp = pltpu.async_copy(pred_hbm.at[row0], pred0, sp0)
    ct = pltpu.async_copy(targ_hbm.at[row0], targ0, st0)
    acc = jnp.zeros((L,), jnp.int32)
    for j in range(ROWS_PER_W):
        b = j % 2
        nb = (j + 1) % 2
        if j + 1 < ROWS_PER_W:
            ncp = pltpu.async_copy(
                pred_hbm.at[row0 + j + 1], preds[nb], sems_p[nb]
            )
            nct = pltpu.async_copy(
                targ_hbm.at[row0 + j + 1], targs[nb], sems_t[nb]
            )
        cp.wait()
        ct.wait()
        acc = _row_correct(preds[b], targs[b], acc)
        if j + 1 < ROWS_PER_W:
            cp, ct = ncp, nct
    out_v[...] = acc
    pltpu.sync_copy(out_v, out_hbm.at[wid])


@jax.jit
def kernel(pred, target):
    mesh = plsc.VectorSubcoreMesh(
        core_axis_name="c", subcore_axis_name="s", num_cores=NC, num_subcores=NS
    )
    partials = pl.kernel(
        _sc_body,
        out_type=jax.ShapeDtypeStruct((NW, L), jnp.int32),
        mesh=mesh,
        scratch_types=[
            pltpu.VMEM((N,), jnp.float32),
            pltpu.VMEM((N,), jnp.float32),
            pltpu.VMEM((N,), jnp.int32),
            pltpu.VMEM((N,), jnp.int32),
            pltpu.VMEM((L,), jnp.int32),
            pltpu.SemaphoreType.DMA,
            pltpu.SemaphoreType.DMA,
            pltpu.SemaphoreType.DMA,
            pltpu.SemaphoreType.DMA,
        ],
        compiler_params=pltpu.CompilerParams(needs_layout_passes=False),
    )(pred, target)
    correct = jnp.sum(partials).astype(jnp.float32)
    return correct * jnp.float32(100.0 / (B * K))


# speedup vs baseline: 1.3861x; 1.1435x over previous
import jax
import jax.numpy as jnp
from jax import lax
from jax.experimental import pallas as pl
from jax.experimental.pallas import tpu as pltpu
from jax.experimental.pallas import tpu_sc as plsc

B = 128
N = 8192
K = 5
L = 16
NC = 2
NS = 16
NW = NC * NS
SC_ROWS = 64
TC_ROWS = B - SC_ROWS
ROWS_PER_W = SC_ROWS // NW
NV = N // L
NEG_INF = float("-inf")


def _row_correct(pred_v, targ_v, acc):
    @plsc.parallel_loop(
        0,
        NV,
        unroll=8,
        carry=tuple(jnp.full((L,), NEG_INF, jnp.float32) for _ in range(K)),
    )
    def ms(i, ms):
        v = pred_v[pl.ds(i * L, L)]
        out = []
        for m in ms:
            hi = jnp.maximum(m, v)
            v = jnp.minimum(m, v)
            out.append(hi)
        return tuple(out)

    def body_t(_, carry):
        c1, c2, c3, c4, c5, accn, cgt, t = carry
        cmax = jnp.maximum(
            jnp.maximum(jnp.maximum(c1, c2), jnp.maximum(c3, c4)), c5
        )
        g = jnp.max(cmax)
        eqs = (
            (c1 == g).astype(jnp.int32)
            + (c2 == g).astype(jnp.int32)
            + (c3 == g).astype(jnp.int32)
            + (c4 == g).astype(jnp.int32)
            + (c5 == g).astype(jnp.int32)
        )
        cnt = jnp.sum(eqs)
        nd = accn < K
        t = jnp.where(nd, g, t)
        cgt = jnp.where(nd, accn, cgt)
        accn = jnp.where(nd, accn + cnt, accn)
        c1 = jnp.where(c1 == g, NEG_INF, c1)
        c2 = jnp.where(c2 == g, NEG_INF, c2)
        c3 = jnp.where(c3 == g, NEG_INF, c3)
        c4 = jnp.where(c4 == g, NEG_INF, c4)
        c5 = jnp.where(c5 == g, NEG_INF, c5)
        return c1, c2, c3, c4, c5, accn, cgt, t

    carry = (*ms, jnp.int32(0), jnp.int32(0), jnp.float32(0))
    carry = lax.fori_loop(0, K, body_t, carry)
    cgt, t = carry[6], carry[7]
    r = K - cgt

    zeros = jnp.zeros((L,), jnp.int32)

    @plsc.parallel_loop(0, NV, unroll=8, carry=(zeros, zeros))
    def accs(i, carry):
        acc_ge, cnt_eq = carry
        v = pred_v[pl.ds(i * L, L)]
        tv = targ_v[pl.ds(i * L, L)]
        acc_ge = acc_ge + jnp.where(v >= t, tv, 0)
        cnt_eq = cnt_eq + (v == t).astype(jnp.int32)
        return acc_ge, cnt_eq

    acc_ge, cnt_eq = accs
    tot_eq = jnp.sum(cnt_eq)

    def tie_scan(_):
        def body_c(i, carry):
            acc_gt, stie, taken = carry
            v = pred_v[pl.ds(i * L, L)]
            tv = targ_v[pl.ds(i * L, L)]
            acc_gt = acc_gt + jnp.where(v > t, tv, 0)
            eqi = (v == t).astype(jnp.int32)
            pref = jnp.cumsum(eqi)
            take = jnp.logical_and(v == t, (taken + pref) <= r)
            stie = stie + jnp.sum(jnp.where(take, tv, 0))
            taken = taken + jnp.sum(eqi)
            return acc_gt, stie, taken

        acc_gt, stie, _ = lax.fori_loop(
            0, NV, body_c, (zeros, jnp.int32(0), jnp.int32(0))
        )
        lane0 = lax.iota(jnp.int32, L) == 0
        return acc_gt + jnp.where(lane0, stie, 0)

    def ge_whole(_):
        return acc_ge

    return acc + lax.cond(tot_eq == r, ge_whole, tie_scan, 0)


def _sc_body(pred_hbm, targ_hbm, out_hbm, pred0, pred1, targ0, targ1, out_v,
             sp0, sp1, st0, st1):
    wid = lax.axis_index("s") * NC + lax.axis_index("c")
    row0 = wid * ROWS_PER_W
    preds = (pred0, pred1)
    targs = (targ0, targ1)
    sems_p = (sp0, sp1)
    sems_t = (st0, st1)
    cp = pltpu.async_copy(pred_hbm.at[row0], pred0, sp0)
    ct = pltpu.async_copy(targ_hbm.at[row0], targ0, st0)
    acc = jnp.zeros((L,), jnp.int32)
    for j in range(ROWS_PER_W):
        b = j % 2
        nb = (j + 1) % 2
        if j + 1 < ROWS_PER_W:
            ncp = pltpu.async_copy(
                pred_hbm.at[row0 + j + 1], preds[nb], sems_p[nb]
            )
            nct = pltpu.async_copy(
                targ_hbm.at[row0 + j + 1], targs[nb], sems_t[nb]
            )
        cp.wait()
        ct.wait()
        acc = _row_correct(preds[b], targs[b], acc)
        if j + 1 < ROWS_PER_W:
            cp, ct = ncp, nct
    out_v[...] = acc
    pltpu.sync_copy(out_v, out_hbm.at[wid])


def _tc_body(pred_ref, targ_ref, out_ref, work_ref):
    work_ref[...] = pred_ref[...]
    zc = jnp.zeros((TC_ROWS, 1), jnp.int32)
    accn, cgt, t = zc, zc, jnp.zeros((TC_ROWS, 1), jnp.float32)
    for _ in range(K):
        w = work_ref[...]
        g = jnp.max(w, axis=1, keepdims=True)
        eq = w == g
        cnt = jnp.sum(eq.astype(jnp.int32), axis=1, keepdims=True)
        nd = accn < K
        t = jnp.where(nd, g, t)
        cgt = jnp.where(nd, accn, cgt)
        accn = jnp.where(nd, accn + cnt, accn)
        work_ref[...] = jnp.where(eq, NEG_INF, w)
    r = K - cgt

    p = pred_ref[...]
    tv = targ_ref[...]
    ge_sum = jnp.sum(jnp.where(p >= t, tv, 0), axis=1, keepdims=True)
    eq_cnt = jnp.sum((p == t).astype(jnp.int32), axis=1, keepdims=True)
    conflict = jnp.any(eq_cnt != r)

    def ordered(_):
        colid = lax.broadcasted_iota(jnp.int32, (TC_ROWS, N), 1)
        work_ref[...] = p
        acc = jnp.zeros((TC_ROWS, 1), jnp.int32)
        for _ in range(K):
            w = work_ref[...]
            g = jnp.max(w, axis=1, keepdims=True)
            eq = w == g
            fi = jnp.min(jnp.where(eq, colid, N), axis=1, keepdims=True)
            sel = colid == fi
            acc = acc + jnp.sum(jnp.where(sel, tv, 0), axis=1, keepdims=True)
            work_ref[...] = jnp.where(sel, NEG_INF, w)
        return acc

    def clean(_):
        return ge_sum

    rows = lax.cond(conflict, ordered, clean, 0)
    total = jnp.sum(rows)
    rid = lax.broadcasted_iota(jnp.int32, (8, 128), 0)
    cid = lax.broadcasted_iota(jnp.int32, (8, 128), 1)
    out_ref[...] = jnp.where((rid == 0) & (cid == 0), total, 0)


@jax.jit
def kernel(pred, target):
    mesh = plsc.VectorSubcoreMesh(
        core_axis_name="c", subcore_axis_name="s", num_cores=NC, num_subcores=NS
    )
    sc_partials = pl.kernel(
        _sc_body,
        out_type=jax.ShapeDtypeStruct((NW, L), jnp.int32),
        mesh=mesh,
        scratch_types=[
            pltpu.VMEM((N,), jnp.float32),
            pltpu.VMEM((N,), jnp.float32),
            pltpu.VMEM((N,), jnp.int32),
            pltpu.VMEM((N,), jnp.int32),
            pltpu.VMEM((L,), jnp.int32),
            pltpu.SemaphoreType.DMA,
            pltpu.SemaphoreType.DMA,
            pltpu.SemaphoreType.DMA,
            pltpu.SemaphoreType.DMA,
        ],
        compiler_params=pltpu.CompilerParams(needs_layout_passes=False),
    )(pred, target)
    tc_out = pl.pallas_call(
        _tc_body,
        out_shape=jax.ShapeDtypeStruct((8, 128), jnp.int32),
        grid=(1,),
        in_specs=[
            pl.BlockSpec((TC_ROWS, N), lambda i: (1, 0)),
            pl.BlockSpec((TC_ROWS, N), lambda i: (1, 0)),
        ],
        out_specs=pl.BlockSpec((8, 128), lambda i: (0, 0)),
        scratch_shapes=[pltpu.VMEM((TC_ROWS, N), jnp.float32)],
    )(pred, target)
    correct = (jnp.sum(sc_partials) + tc_out[0, 0]).astype(jnp.float32)
    return correct * jnp.float32(100.0 / (B * K))
